# baseline (speedup 1.0000x reference)
import functools
import math

import jax
import jax.numpy as jnp
from jax import lax
from jax.experimental import pallas as pl
from jax.experimental.pallas import tpu as pltpu
from jax.experimental.pallas import tpu_sc as plsc

N = 10000
E = 320000
D = 128
H = 128
C = 64
K = 10

NC = 2
NS = 16
NW = NC * NS

CH = 128
NBUF = 4
LOOKAHEAD = 2
NPAD = 10240
RPS = NPAD // NS
EPAD = ((E + NW * CH * NBUF - 1) // (NW * CH * NBUF)) * (NW * CH * NBUF)
EPT = EPAD // NW
NCHK = EPT // CH

ROW_BLK = 512
MLP_BLK = 256


def _sc_body(g_hbm, src_hbm, dst_hbm, zero_hbm, out_hbm,
             src_v, dst_v, rows_v, acc_sh, gsem, ssem):
    cid = lax.axis_index("c")
    sid = lax.axis_index("s")
    wid = sid * NC + cid

    pltpu.sync_copy(src_hbm.at[wid], src_v)
    pltpu.sync_copy(dst_hbm.at[wid], dst_v)

    rbase = sid * RPS
    pltpu.sync_copy(zero_hbm.at[pl.ds(rbase, RPS)], acc_sh.at[pl.ds(rbase, RPS)])
    plsc.subcore_barrier()

    gh = [None] * NCHK
    sh = [None] * NCHK
    nxt = 0
    for j in range(NCHK):
        while nxt <= min(j + LOOKAHEAD, NCHK - 1):
            b = nxt % NBUF
            if nxt >= NBUF:
                sh[nxt - NBUF].wait()
            gh[nxt] = pltpu.async_copy(
                g_hbm.at[src_v.at[nxt]], rows_v.at[b], gsem.at[b])
            nxt += 1
        gh[j].wait()
        sh[j] = pltpu.async_copy(
            rows_v.at[j % NBUF], acc_sh.at[dst_v.at[j]], ssem.at[j % NBUF],
            add=True)
    for j in range(NCHK - NBUF, NCHK):
        sh[j].wait()

    plsc.subcore_barrier()
    pltpu.sync_copy(acc_sh.at[pl.ds(rbase, RPS)],
                    out_hbm.at[cid, pl.ds(rbase, RPS)])


def _make_sc_spmm():
    mesh = plsc.VectorSubcoreMesh(core_axis_name="c", subcore_axis_name="s")
    return pl.kernel(
        _sc_body,
        mesh=mesh,
        compiler_params=pltpu.CompilerParams(use_tc_tiling_on_sc=False),
        out_type=jax.ShapeDtypeStruct((NC, NPAD, C), jnp.float32),
        scratch_types=[
            pltpu.VMEM((NCHK, CH), jnp.int32),
            pltpu.VMEM((NCHK, CH), jnp.int32),
            pltpu.VMEM((NBUF, CH, C), jnp.float32),
            pltpu.VMEM_SHARED((NPAD, C), jnp.float32),
            pltpu.SemaphoreType.DMA((NBUF,)),
            pltpu.SemaphoreType.DMA((NBUF,)),
        ],
    )


def _mlp_body(x_ref, w1_ref, b1_ref, w2_ref, b2_ref, o_ref):
    h = jnp.dot(x_ref[...], w1_ref[...], preferred_element_type=jnp.float32)
    h = jnp.maximum(h + b1_ref[...], 0.0)
    o_ref[...] = (
        jnp.dot(h, w2_ref[...], preferred_element_type=jnp.float32) + b2_ref[...]
    )


def _mlp(xp, W1, b1, W2, b2):
    grid = (NPAD // MLP_BLK,)
    return pl.pallas_call(
        _mlp_body,
        grid=grid,
        in_specs=[
            pl.BlockSpec((MLP_BLK, D), lambda i: (i, 0)),
            pl.BlockSpec((D, H), lambda i: (0, 0)),
            pl.BlockSpec((1, H), lambda i: (0, 0)),
            pl.BlockSpec((H, C), lambda i: (0, 0)),
            pl.BlockSpec((1, C), lambda i: (0, 0)),
        ],
        out_specs=pl.BlockSpec((MLP_BLK, C), lambda i: (i, 0)),
        out_shape=jax.ShapeDtypeStruct((NPAD, C), jnp.float32),
    )(xp, W1, b1.reshape(1, H), W2, b2.reshape(1, C))


def _norm_body(temp_ref, dacc_ref, h0_ref, dinv_ref, g0_ref, s0_ref, gs0_ref):
    deg = dacc_ref[0] + dacc_ref[1]
    dinv = jnp.where(deg > 0.0, lax.rsqrt(jnp.maximum(deg, 1e-12)), 0.0)
    cktk = (1.0 / 2.0**K) * jnp.maximum(temp_ref[K], 0.0)
    h0 = h0_ref[...]
    g0 = dinv * h0
    dinv_ref[...] = dinv
    g0_ref[...] = g0
    s0_ref[...] = cktk * h0
    gs0_ref[...] = cktk * g0


def _norm(temp, dacc, h0):
    grid = (NPAD // ROW_BLK,)
    fs = jax.ShapeDtypeStruct((NPAD, C), jnp.float32)
    return pl.pallas_call(
        _norm_body,
        grid=grid,
        in_specs=[
            pl.BlockSpec(memory_space=pltpu.SMEM),
            pl.BlockSpec((2, ROW_BLK, C), lambda i: (0, i, 0)),
            pl.BlockSpec((ROW_BLK, C), lambda i: (i, 0)),
        ],
        out_specs=[pl.BlockSpec((ROW_BLK, C), lambda i: (i, 0))] * 4,
        out_shape=[fs, fs, fs, fs],
    )(temp, dacc, h0)


def _comb_body(temp_ref, h_ref, acc_ref, t_ref, dinv_ref, ho_ref, go_ref,
               *, beta, cm, m):
    dinv = dinv_ref[...]
    hn = h_ref[...] + beta * (dinv * (acc_ref[0] + acc_ref[1]))
    if cm != 0.0:
        hn = hn + (cm * jnp.maximum(temp_ref[m], 0.0)) * t_ref[...]
    ho_ref[...] = hn
    go_ref[...] = dinv * hn


def _combine(temp, h, acc, t, dinv, *, beta, cm, m):
    grid = (NPAD // ROW_BLK,)
    fs = jax.ShapeDtypeStruct((NPAD, C), jnp.float32)
    return pl.pallas_call(
        functools.partial(_comb_body, beta=beta, cm=cm, m=m),
        grid=grid,
        in_specs=[
            pl.BlockSpec(memory_space=pltpu.SMEM),
            pl.BlockSpec((ROW_BLK, C), lambda i: (i, 0)),
            pl.BlockSpec((2, ROW_BLK, C), lambda i: (0, i, 0)),
            pl.BlockSpec((ROW_BLK, C), lambda i: (i, 0)),
            pl.BlockSpec((ROW_BLK, C), lambda i: (i, 0)),
        ],
        out_specs=[pl.BlockSpec((ROW_BLK, C), lambda i: (i, 0))] * 2,
        out_shape=[fs, fs],
    )(temp, h, acc, t, dinv)


def _lsm_body(x_ref, o_ref):
    x = x_ref[...]
    mx = jnp.max(x, axis=1, keepdims=True)
    ex = jnp.exp(x - mx)
    lse = jnp.log(jnp.sum(ex, axis=1, keepdims=True))
    o_ref[...] = x - mx - lse


def _log_softmax(s):
    grid = (NPAD // ROW_BLK,)
    return pl.pallas_call(
        _lsm_body,
        grid=grid,
        in_specs=[pl.BlockSpec((ROW_BLK, C), lambda i: (i, 0))],
        out_specs=pl.BlockSpec((ROW_BLK, C), lambda i: (i, 0)),
        out_shape=jax.ShapeDtypeStruct((NPAD, C), jnp.float32),
    )(s)


def kernel(x, edge_index, W1, b1, W2, b2, temp):
    xp = jnp.zeros((NPAD, D), jnp.float32).at[:N].set(x)

    pad_e = EPAD - E
    fill = jnp.full((pad_e,), N, jnp.int32)
    srcp = jnp.concatenate([edge_index[0], fill]).reshape(NW, NCHK, CH)
    dstp = jnp.concatenate([edge_index[1], fill]).reshape(NW, NCHK, CH)

    row_valid = (jnp.arange(NPAD, dtype=jnp.int32) < N).astype(jnp.float32)
    ones_feat = jnp.broadcast_to(row_valid[:, None], (NPAD, C))
    zero_feat = jnp.zeros((NPAD, C), jnp.float32)

    sc_spmm = _make_sc_spmm()

    def spmm(g):
        return sc_spmm(g, srcp, dstp, zero_feat)

    h0 = _mlp(xp, W1, b1, W2, b2)

    dacc = spmm(ones_feat)
    dinv, g, s, gs = _norm(temp, dacc, h0)

    ccoef = [math.comb(K, m) / 2.0**K for m in range(K + 1)]

    tmps = [h0]
    h = h0
    for _ in range(K):
        acc = spmm(g)
        h, g = _combine(temp, h, acc, h, dinv, beta=1.0, cm=0.0, m=0)
        tmps.append(h)

    for m in range(K - 1, -1, -1):
        acc = spmm(gs)
        s, gs = _combine(temp, s, acc, tmps[K - m], dinv,
                         beta=-1.0, cm=ccoef[m], m=m)

    out = _log_softmax(s)
    return out[:N]


# speedup vs baseline: 1.8160x; 1.8160x over previous
import functools
import math

import numpy as np
import jax
import jax.numpy as jnp
from jax import lax
from jax.experimental import pallas as pl
from jax.experimental.pallas import tpu as pltpu
from jax.experimental.pallas import tpu_sc as plsc

N = 10000
E = 320000
D = 128
H = 128
C = 64
K = 10

NC = 2
NS = 16
NW = NC * NS

CH = 128
NBUF = 4
LOOKAHEAD = 2
NPAD = 10240
RPS = NPAD // NS
EPAD = ((E + NW * CH * NBUF - 1) // (NW * CH * NBUF)) * (NW * CH * NBUF)
EPT = EPAD // NW
NCHK = EPT // CH

ROW_BLK = 512
MLP_BLK = 256


def _bern_monomial_basis():
    rows = []
    for m in range(K + 1):
        a = np.array([1.0])
        for _ in range(m):
            a = np.convolve(a, [1.0, -1.0])
        for _ in range(K - m):
            a = np.convolve(a, [1.0, 1.0])
        rows.append(math.comb(K, m) / 2.0**K * a)
    return np.stack(rows)


_BMAT = _bern_monomial_basis()


def _sc_body(g_hbm, src_hbm, dst_hbm, zero_hbm, out_hbm,
             src_v, dst_v, rows_v, acc_sh, gsem):
    cid = lax.axis_index("c")
    sid = lax.axis_index("s")
    wid = sid * NC + cid

    pltpu.sync_copy(src_hbm.at[wid], src_v)
    pltpu.sync_copy(dst_hbm.at[wid], dst_v)

    rbase = sid * RPS
    pltpu.sync_copy(zero_hbm.at[pl.ds(rbase, RPS)], acc_sh.at[pl.ds(rbase, RPS)])
    plsc.subcore_barrier()

    def chunk(j, carry):
        pltpu.async_copy(g_hbm.at[src_v.at[j]], rows_v, gsem).wait()
        pltpu.sync_copy(rows_v, acc_sh.at[dst_v.at[j]], add=True)
        return carry

    lax.fori_loop(0, NCHK, chunk, 0)

    plsc.subcore_barrier()
    pltpu.sync_copy(acc_sh.at[pl.ds(rbase, RPS)],
                    out_hbm.at[cid, pl.ds(rbase, RPS)])


def _make_sc_spmm():
    mesh = plsc.VectorSubcoreMesh(core_axis_name="c", subcore_axis_name="s")
    return pl.kernel(
        _sc_body,
        mesh=mesh,
        compiler_params=pltpu.CompilerParams(use_tc_tiling_on_sc=False),
        out_type=jax.ShapeDtypeStruct((NC, NPAD, C), jnp.float32),
        scratch_types=[
            pltpu.VMEM((NCHK, CH), jnp.int32),
            pltpu.VMEM((NCHK, CH), jnp.int32),
            pltpu.VMEM((CH, C), jnp.float32),
            pltpu.VMEM_SHARED((NPAD, C), jnp.float32),
            pltpu.SemaphoreType.DMA,
        ],
    )


def _mlp_body(x_ref, w1_ref, b1_ref, w2_ref, b2_ref, o_ref):
    h = jnp.dot(x_ref[...], w1_ref[...], preferred_element_type=jnp.float32)
    h = jnp.maximum(h + b1_ref[...], 0.0)
    o_ref[...] = (
        jnp.dot(h, w2_ref[...], preferred_element_type=jnp.float32) + b2_ref[...]
    )


def _mlp(xp, W1, b1, W2, b2):
    grid = (NPAD // MLP_BLK,)
    return pl.pallas_call(
        _mlp_body,
        grid=grid,
        in_specs=[
            pl.BlockSpec((MLP_BLK, D), lambda i: (i, 0)),
            pl.BlockSpec((D, H), lambda i: (0, 0)),
            pl.BlockSpec((1, H), lambda i: (0, 0)),
            pl.BlockSpec((H, C), lambda i: (0, 0)),
            pl.BlockSpec((1, C), lambda i: (0, 0)),
        ],
        out_specs=pl.BlockSpec((MLP_BLK, C), lambda i: (i, 0)),
        out_shape=jax.ShapeDtypeStruct((NPAD, C), jnp.float32),
    )(xp, W1, b1.reshape(1, H), W2, b2.reshape(1, C))


def _poly_coeff(temp_ref, k):
    pk = 0.0
    for m in range(K + 1):
        b = float(_BMAT[m, k])
        if b != 0.0:
            pk = pk + b * jnp.maximum(temp_ref[m], 0.0)
    return pk


def _norm_body(temp_ref, dacc_ref, h0_ref, dinv_ref, r0_ref, g0_ref):
    deg = dacc_ref[0] + dacc_ref[1]
    dinv = jnp.where(deg > 0.0, lax.rsqrt(jnp.maximum(deg, 1e-12)), 0.0)
    pK = _poly_coeff(temp_ref, K)
    r0 = pK * h0_ref[...]
    dinv_ref[...] = dinv
    r0_ref[...] = r0
    g0_ref[...] = dinv * r0


def _norm(temp, dacc, h0):
    grid = (NPAD // ROW_BLK,)
    fs = jax.ShapeDtypeStruct((NPAD, C), jnp.float32)
    return pl.pallas_call(
        _norm_body,
        grid=grid,
        in_specs=[
            pl.BlockSpec(memory_space=pltpu.SMEM),
            pl.BlockSpec((2, ROW_BLK, C), lambda i: (0, i, 0)),
            pl.BlockSpec((ROW_BLK, C), lambda i: (i, 0)),
        ],
        out_specs=[pl.BlockSpec((ROW_BLK, C), lambda i: (i, 0))] * 3,
        out_shape=[fs, fs, fs],
    )(temp, dacc, h0)


def _comb_body(temp_ref, acc_ref, h0_ref, dinv_ref, ro_ref, go_ref, *, k):
    dinv = dinv_ref[...]
    pk = _poly_coeff(temp_ref, k)
    rn = dinv * (acc_ref[0] + acc_ref[1]) + pk * h0_ref[...]
    ro_ref[...] = rn
    go_ref[...] = dinv * rn


def _combine(temp, acc, h0, dinv, *, k):
    grid = (NPAD // ROW_BLK,)
    fs = jax.ShapeDtypeStruct((NPAD, C), jnp.float32)
    return pl.pallas_call(
        functools.partial(_comb_body, k=k),
        grid=grid,
        in_specs=[
            pl.BlockSpec(memory_space=pltpu.SMEM),
            pl.BlockSpec((2, ROW_BLK, C), lambda i: (0, i, 0)),
            pl.BlockSpec((ROW_BLK, C), lambda i: (i, 0)),
            pl.BlockSpec((ROW_BLK, C), lambda i: (i, 0)),
        ],
        out_specs=[pl.BlockSpec((ROW_BLK, C), lambda i: (i, 0))] * 2,
        out_shape=[fs, fs],
    )(temp, acc, h0, dinv)


def _lsm_body(x_ref, o_ref):
    x = x_ref[...]
    mx = jnp.max(x, axis=1, keepdims=True)
    ex = jnp.exp(x - mx)
    lse = jnp.log(jnp.sum(ex, axis=1, keepdims=True))
    o_ref[...] = x - mx - lse


def _log_softmax(s):
    grid = (NPAD // ROW_BLK,)
    return pl.pallas_call(
        _lsm_body,
        grid=grid,
        in_specs=[pl.BlockSpec((ROW_BLK, C), lambda i: (i, 0))],
        out_specs=pl.BlockSpec((ROW_BLK, C), lambda i: (i, 0)),
        out_shape=jax.ShapeDtypeStruct((NPAD, C), jnp.float32),
    )(s)


def kernel(x, edge_index, W1, b1, W2, b2, temp):
    xp = jnp.zeros((NPAD, D), jnp.float32).at[:N].set(x)

    pad_e = EPAD - E
    fill = jnp.full((pad_e,), N, jnp.int32)
    srcp = jnp.concatenate([edge_index[0], fill]).reshape(NW, NCHK, CH)
    dstp = jnp.concatenate([edge_index[1], fill]).reshape(NW, NCHK, CH)

    row_valid = (jnp.arange(NPAD, dtype=jnp.int32) < N).astype(jnp.float32)
    ones_feat = jnp.broadcast_to(row_valid[:, None], (NPAD, C))
    zero_feat = jnp.zeros((NPAD, C), jnp.float32)

    sc_spmm = _make_sc_spmm()

    def spmm(g):
        return sc_spmm(g, srcp, dstp, zero_feat)

    h0 = _mlp(xp, W1, b1, W2, b2)

    dacc = spmm(ones_feat)
    dinv, r, g = _norm(temp, dacc, h0)

    for k in range(K - 1, -1, -1):
        acc = spmm(g)
        r, g = _combine(temp, acc, h0, dinv, k=k)

    out = _log_softmax(r)
    return out[:N]


# speedup vs baseline: 3.5675x; 1.9645x over previous
import functools
import math

import numpy as np
import jax
import jax.numpy as jnp
from jax import lax
from jax.experimental import pallas as pl
from jax.experimental.pallas import tpu as pltpu
from jax.experimental.pallas import tpu_sc as plsc

N = 10000
E = 320000
D = 128
H = 128
C = 64
K = 10

NC = 2
NS = 16
NW = NC * NS

CH = 128
NBUF = 4
LOOKAHEAD = 2
NPAD = 10240
RPS = NPAD // NS
EPAD = ((E + NW * CH * NBUF - 1) // (NW * CH * NBUF)) * (NW * CH * NBUF)
EPT = EPAD // NW
NCHK = EPT // CH

ROW_BLK = 512
MLP_BLK = 256


def _bern_monomial_basis():
    rows = []
    for m in range(K + 1):
        a = np.array([1.0])
        for _ in range(m):
            a = np.convolve(a, [1.0, -1.0])
        for _ in range(K - m):
            a = np.convolve(a, [1.0, 1.0])
        rows.append(math.comb(K, m) / 2.0**K * a)
    return np.stack(rows)


_BMAT = _bern_monomial_basis()


def _sc_body(g_hbm, src_hbm, dst_hbm, zero_hbm, out_hbm,
             src_v, dst_v, rows_v, g_sh, acc_sh, gsem):
    cid = lax.axis_index("c")
    sid = lax.axis_index("s")
    wid = sid * NC + cid

    pltpu.sync_copy(src_hbm.at[wid], src_v)
    pltpu.sync_copy(dst_hbm.at[wid], dst_v)

    rbase = sid * RPS
    pltpu.sync_copy(g_hbm.at[pl.ds(rbase, RPS)], g_sh.at[pl.ds(rbase, RPS)])
    pltpu.sync_copy(zero_hbm.at[pl.ds(rbase, RPS)], acc_sh.at[pl.ds(rbase, RPS)])
    plsc.subcore_barrier()

    def chunk(j, carry):
        pltpu.async_copy(g_sh.at[src_v.at[j]], rows_v, gsem).wait()
        pltpu.sync_copy(rows_v, acc_sh.at[dst_v.at[j]], add=True)
        return carry

    lax.fori_loop(0, NCHK, chunk, 0)

    plsc.subcore_barrier()
    pltpu.sync_copy(acc_sh.at[pl.ds(rbase, RPS)],
                    out_hbm.at[cid, pl.ds(rbase, RPS)])


def _make_sc_spmm():
    mesh = plsc.VectorSubcoreMesh(core_axis_name="c", subcore_axis_name="s")
    return pl.kernel(
        _sc_body,
        mesh=mesh,
        compiler_params=pltpu.CompilerParams(use_tc_tiling_on_sc=False),
        out_type=jax.ShapeDtypeStruct((NC, NPAD, C), jnp.float32),
        scratch_types=[
            pltpu.VMEM((NCHK, CH), jnp.int32),
            pltpu.VMEM((NCHK, CH), jnp.int32),
            pltpu.VMEM((CH, C), jnp.float32),
            pltpu.VMEM_SHARED((NPAD, C), jnp.float32),
            pltpu.VMEM_SHARED((NPAD, C), jnp.float32),
            pltpu.SemaphoreType.DMA,
        ],
    )


def _mlp_body(x_ref, w1_ref, b1_ref, w2_ref, b2_ref, o_ref):
    h = jnp.dot(x_ref[...], w1_ref[...], preferred_element_type=jnp.float32)
    h = jnp.maximum(h + b1_ref[...], 0.0)
    o_ref[...] = (
        jnp.dot(h, w2_ref[...], preferred_element_type=jnp.float32) + b2_ref[...]
    )


def _mlp(xp, W1, b1, W2, b2):
    grid = (NPAD // MLP_BLK,)
    return pl.pallas_call(
        _mlp_body,
        grid=grid,
        in_specs=[
            pl.BlockSpec((MLP_BLK, D), lambda i: (i, 0)),
            pl.BlockSpec((D, H), lambda i: (0, 0)),
            pl.BlockSpec((1, H), lambda i: (0, 0)),
            pl.BlockSpec((H, C), lambda i: (0, 0)),
            pl.BlockSpec((1, C), lambda i: (0, 0)),
        ],
        out_specs=pl.BlockSpec((MLP_BLK, C), lambda i: (i, 0)),
        out_shape=jax.ShapeDtypeStruct((NPAD, C), jnp.float32),
    )(xp, W1, b1.reshape(1, H), W2, b2.reshape(1, C))


def _poly_coeff(temp_ref, k):
    pk = 0.0
    for m in range(K + 1):
        b = float(_BMAT[m, k])
        if b != 0.0:
            pk = pk + b * jnp.maximum(temp_ref[m], 0.0)
    return pk


def _norm_body(temp_ref, dacc_ref, h0_ref, dinv_ref, r0_ref, g0_ref):
    deg = dacc_ref[0] + dacc_ref[1]
    dinv = jnp.where(deg > 0.0, lax.rsqrt(jnp.maximum(deg, 1e-12)), 0.0)
    pK = _poly_coeff(temp_ref, K)
    r0 = pK * h0_ref[...]
    dinv_ref[...] = dinv
    r0_ref[...] = r0
    g0_ref[...] = dinv * r0


def _norm(temp, dacc, h0):
    grid = (NPAD // ROW_BLK,)
    fs = jax.ShapeDtypeStruct((NPAD, C), jnp.float32)
    return pl.pallas_call(
        _norm_body,
        grid=grid,
        in_specs=[
            pl.BlockSpec(memory_space=pltpu.SMEM),
            pl.BlockSpec((2, ROW_BLK, C), lambda i: (0, i, 0)),
            pl.BlockSpec((ROW_BLK, C), lambda i: (i, 0)),
        ],
        out_specs=[pl.BlockSpec((ROW_BLK, C), lambda i: (i, 0))] * 3,
        out_shape=[fs, fs, fs],
    )(temp, dacc, h0)


def _comb_body(temp_ref, acc_ref, h0_ref, dinv_ref, ro_ref, go_ref, *, k):
    dinv = dinv_ref[...]
    pk = _poly_coeff(temp_ref, k)
    rn = dinv * (acc_ref[0] + acc_ref[1]) + pk * h0_ref[...]
    ro_ref[...] = rn
    go_ref[...] = dinv * rn


def _combine(temp, acc, h0, dinv, *, k):
    grid = (NPAD // ROW_BLK,)
    fs = jax.ShapeDtypeStruct((NPAD, C), jnp.float32)
    return pl.pallas_call(
        functools.partial(_comb_body, k=k),
        grid=grid,
        in_specs=[
            pl.BlockSpec(memory_space=pltpu.SMEM),
            pl.BlockSpec((2, ROW_BLK, C), lambda i: (0, i, 0)),
            pl.BlockSpec((ROW_BLK, C), lambda i: (i, 0)),
            pl.BlockSpec((ROW_BLK, C), lambda i: (i, 0)),
        ],
        out_specs=[pl.BlockSpec((ROW_BLK, C), lambda i: (i, 0))] * 2,
        out_shape=[fs, fs],
    )(temp, acc, h0, dinv)


def _lsm_body(x_ref, o_ref):
    x = x_ref[...]
    mx = jnp.max(x, axis=1, keepdims=True)
    ex = jnp.exp(x - mx)
    lse = jnp.log(jnp.sum(ex, axis=1, keepdims=True))
    o_ref[...] = x - mx - lse


def _log_softmax(s):
    grid = (NPAD // ROW_BLK,)
    return pl.pallas_call(
        _lsm_body,
        grid=grid,
        in_specs=[pl.BlockSpec((ROW_BLK, C), lambda i: (i, 0))],
        out_specs=pl.BlockSpec((ROW_BLK, C), lambda i: (i, 0)),
        out_shape=jax.ShapeDtypeStruct((NPAD, C), jnp.float32),
    )(s)


def kernel(x, edge_index, W1, b1, W2, b2, temp):
    xp = jnp.zeros((NPAD, D), jnp.float32).at[:N].set(x)

    pad_e = EPAD - E
    fill = jnp.full((pad_e,), N, jnp.int32)
    srcp = jnp.concatenate([edge_index[0], fill]).reshape(NW, NCHK, CH)
    dstp = jnp.concatenate([edge_index[1], fill]).reshape(NW, NCHK, CH)

    row_valid = (jnp.arange(NPAD, dtype=jnp.int32) < N).astype(jnp.float32)
    ones_feat = jnp.broadcast_to(row_valid[:, None], (NPAD, C))
    zero_feat = jnp.zeros((NPAD, C), jnp.float32)

    sc_spmm = _make_sc_spmm()

    def spmm(g):
        return sc_spmm(g, srcp, dstp, zero_feat)

    h0 = _mlp(xp, W1, b1, W2, b2)

    dacc = spmm(ones_feat)
    dinv, r, g = _norm(temp, dacc, h0)

    for k in range(K - 1, -1, -1):
        acc = spmm(g)
        r, g = _combine(temp, acc, h0, dinv, k=k)

    out = _log_softmax(r)
    return out[:N]


# speedup vs baseline: 4.4303x; 1.2419x over previous
import functools
import math

import numpy as np
import jax
import jax.numpy as jnp
from jax import lax
from jax.experimental import pallas as pl
from jax.experimental.pallas import tpu as pltpu
from jax.experimental.pallas import tpu_sc as plsc

N = 10000
E = 320000
D = 128
H = 128
C = 64
K = 10

NC = 2
NS = 16
NW = NC * NS

CH = 128
NBUF = 4
LOOKAHEAD = 2
NPAD = 10240
RPS = NPAD // NS
EPAD = ((E + NW * CH * NBUF - 1) // (NW * CH * NBUF)) * (NW * CH * NBUF)
EPT = EPAD // NW
NCHK = EPT // CH

ROW_BLK = 512
MLP_BLK = 256


def _bern_monomial_basis():
    rows = []
    for m in range(K + 1):
        a = np.array([1.0])
        for _ in range(m):
            a = np.convolve(a, [1.0, -1.0])
        for _ in range(K - m):
            a = np.convolve(a, [1.0, 1.0])
        rows.append(math.comb(K, m) / 2.0**K * a)
    return np.stack(rows)


_BMAT = _bern_monomial_basis()


def _sc_body(g_hbm, src_hbm, dst_hbm, zero_hbm, out_hbm,
             src_v, dst_v, rows_v, g_sh, acc_sh, gsem):
    cid = lax.axis_index("c")
    sid = lax.axis_index("s")
    wid = sid * NC + cid

    pltpu.sync_copy(src_hbm.at[wid], src_v)
    pltpu.sync_copy(dst_hbm.at[wid], dst_v)

    rbase = sid * RPS
    pltpu.sync_copy(g_hbm.at[pl.ds(rbase, RPS)], g_sh.at[pl.ds(rbase, RPS)])
    pltpu.sync_copy(zero_hbm.at[pl.ds(rbase, RPS)], acc_sh.at[pl.ds(rbase, RPS)])
    plsc.subcore_barrier()

    gh = [None, None]
    gh[0] = pltpu.async_copy(g_sh.at[src_v.at[0]], rows_v.at[0], gsem.at[0])
    for j in range(NCHK):
        b = j % 2
        if j + 1 < NCHK:
            gh[1 - b] = pltpu.async_copy(
                g_sh.at[src_v.at[j + 1]], rows_v.at[1 - b], gsem.at[1 - b])
        gh[b].wait()
        pltpu.sync_copy(rows_v.at[b], acc_sh.at[dst_v.at[j]], add=True)

    plsc.subcore_barrier()
    pltpu.sync_copy(acc_sh.at[pl.ds(rbase, RPS)],
                    out_hbm.at[cid, pl.ds(rbase, RPS)])


def _make_sc_spmm():
    mesh = plsc.VectorSubcoreMesh(core_axis_name="c", subcore_axis_name="s")
    return pl.kernel(
        _sc_body,
        mesh=mesh,
        compiler_params=pltpu.CompilerParams(use_tc_tiling_on_sc=False),
        out_type=jax.ShapeDtypeStruct((NC, NPAD, C), jnp.float32),
        scratch_types=[
            pltpu.VMEM((NCHK, CH), jnp.int32),
            pltpu.VMEM((NCHK, CH), jnp.int32),
            pltpu.VMEM((2, CH, C), jnp.float32),
            pltpu.VMEM_SHARED((NPAD, C), jnp.float32),
            pltpu.VMEM_SHARED((NPAD, C), jnp.float32),
            pltpu.SemaphoreType.DMA((2,)),
        ],
    )


def _mlp_body(x_ref, w1_ref, b1_ref, w2_ref, b2_ref, o_ref):
    h = jnp.dot(x_ref[...], w1_ref[...], preferred_element_type=jnp.float32)
    h = jnp.maximum(h + b1_ref[...], 0.0)
    o_ref[...] = (
        jnp.dot(h, w2_ref[...], preferred_element_type=jnp.float32) + b2_ref[...]
    )


def _mlp(xp, W1, b1, W2, b2):
    grid = (NPAD // MLP_BLK,)
    return pl.pallas_call(
        _mlp_body,
        grid=grid,
        in_specs=[
            pl.BlockSpec((MLP_BLK, D), lambda i: (i, 0)),
            pl.BlockSpec((D, H), lambda i: (0, 0)),
            pl.BlockSpec((1, H), lambda i: (0, 0)),
            pl.BlockSpec((H, C), lambda i: (0, 0)),
            pl.BlockSpec((1, C), lambda i: (0, 0)),
        ],
        out_specs=pl.BlockSpec((MLP_BLK, C), lambda i: (i, 0)),
        out_shape=jax.ShapeDtypeStruct((NPAD, C), jnp.float32),
    )(xp, W1, b1.reshape(1, H), W2, b2.reshape(1, C))


def _poly_coeff(temp_ref, k):
    pk = 0.0
    for m in range(K + 1):
        b = float(_BMAT[m, k])
        if b != 0.0:
            pk = pk + b * jnp.maximum(temp_ref[m], 0.0)
    return pk


def _norm_body(temp_ref, dacc_ref, h0_ref, dinv_ref, r0_ref, g0_ref):
    deg = dacc_ref[0] + dacc_ref[1]
    dinv = jnp.where(deg > 0.0, lax.rsqrt(jnp.maximum(deg, 1e-12)), 0.0)
    pK = _poly_coeff(temp_ref, K)
    r0 = pK * h0_ref[...]
    dinv_ref[...] = dinv
    r0_ref[...] = r0
    g0_ref[...] = dinv * r0


def _norm(temp, dacc, h0):
    grid = (NPAD // ROW_BLK,)
    fs = jax.ShapeDtypeStruct((NPAD, C), jnp.float32)
    return pl.pallas_call(
        _norm_body,
        grid=grid,
        in_specs=[
            pl.BlockSpec(memory_space=pltpu.SMEM),
            pl.BlockSpec((2, ROW_BLK, C), lambda i: (0, i, 0)),
            pl.BlockSpec((ROW_BLK, C), lambda i: (i, 0)),
        ],
        out_specs=[pl.BlockSpec((ROW_BLK, C), lambda i: (i, 0))] * 3,
        out_shape=[fs, fs, fs],
    )(temp, dacc, h0)


def _comb_body(temp_ref, acc_ref, h0_ref, dinv_ref, ro_ref, go_ref, *, k):
    dinv = dinv_ref[...]
    pk = _poly_coeff(temp_ref, k)
    rn = dinv * (acc_ref[0] + acc_ref[1]) + pk * h0_ref[...]
    ro_ref[...] = rn
    go_ref[...] = dinv * rn


def _combine(temp, acc, h0, dinv, *, k):
    grid = (NPAD // ROW_BLK,)
    fs = jax.ShapeDtypeStruct((NPAD, C), jnp.float32)
    return pl.pallas_call(
        functools.partial(_comb_body, k=k),
        grid=grid,
        in_specs=[
            pl.BlockSpec(memory_space=pltpu.SMEM),
            pl.BlockSpec((2, ROW_BLK, C), lambda i: (0, i, 0)),
            pl.BlockSpec((ROW_BLK, C), lambda i: (i, 0)),
            pl.BlockSpec((ROW_BLK, C), lambda i: (i, 0)),
        ],
        out_specs=[pl.BlockSpec((ROW_BLK, C), lambda i: (i, 0))] * 2,
        out_shape=[fs, fs],
    )(temp, acc, h0, dinv)


def _lsm_body(x_ref, o_ref):
    x = x_ref[...]
    mx = jnp.max(x, axis=1, keepdims=True)
    ex = jnp.exp(x - mx)
    lse = jnp.log(jnp.sum(ex, axis=1, keepdims=True))
    o_ref[...] = x - mx - lse


def _log_softmax(s):
    grid = (NPAD // ROW_BLK,)
    return pl.pallas_call(
        _lsm_body,
        grid=grid,
        in_specs=[pl.BlockSpec((ROW_BLK, C), lambda i: (i, 0))],
        out_specs=pl.BlockSpec((ROW_BLK, C), lambda i: (i, 0)),
        out_shape=jax.ShapeDtypeStruct((NPAD, C), jnp.float32),
    )(s)


def kernel(x, edge_index, W1, b1, W2, b2, temp):
    xp = jnp.zeros((NPAD, D), jnp.float32).at[:N].set(x)

    pad_e = EPAD - E
    fill = jnp.full((pad_e,), N, jnp.int32)
    srcp = jnp.concatenate([edge_index[0], fill]).reshape(NW, NCHK, CH)
    dstp = jnp.concatenate([edge_index[1], fill]).reshape(NW, NCHK, CH)

    row_valid = (jnp.arange(NPAD, dtype=jnp.int32) < N).astype(jnp.float32)
    ones_feat = jnp.broadcast_to(row_valid[:, None], (NPAD, C))
    zero_feat = jnp.zeros((NPAD, C), jnp.float32)

    sc_spmm = _make_sc_spmm()

    def spmm(g):
        return sc_spmm(g, srcp, dstp, zero_feat)

    h0 = _mlp(xp, W1, b1, W2, b2)

    dacc = spmm(ones_feat)
    dinv, r, g = _norm(temp, dacc, h0)

    for k in range(K - 1, -1, -1):
        acc = spmm(g)
        r, g = _combine(temp, acc, h0, dinv, k=k)

    out = _log_softmax(r)
    return out[:N]


# speedup vs baseline: 4.8758x; 1.1006x over previous
import functools
import math

import numpy as np
import jax
import jax.numpy as jnp
from jax import lax
from jax.experimental import pallas as pl
from jax.experimental.pallas import tpu as pltpu
from jax.experimental.pallas import tpu_sc as plsc

N = 10000
E = 320000
D = 128
H = 128
C = 64
K = 10

NC = 2
NS = 16
NW = NC * NS

CH = 128
NBUF = 4
LOOKAHEAD = 2
NPAD = 10240
RPS = NPAD // NS
EPAD = ((E + NW * CH * NBUF - 1) // (NW * CH * NBUF)) * (NW * CH * NBUF)
EPT = EPAD // NW
NCHK = EPT // CH

ROW_BLK = 1024
MLP_BLK = 256
CDEG = 16


def _bern_monomial_basis():
    rows = []
    for m in range(K + 1):
        a = np.array([1.0])
        for _ in range(m):
            a = np.convolve(a, [1.0, -1.0])
        for _ in range(K - m):
            a = np.convolve(a, [1.0, 1.0])
        rows.append(math.comb(K, m) / 2.0**K * a)
    return np.stack(rows)


_BMAT = _bern_monomial_basis()


def _sc_body(g_hbm, src_hbm, dst_hbm, zero_hbm, out_hbm,
             src_v, dst_v, rows_v, g_sh, acc_sh, gsem):
    cid = lax.axis_index("c")
    sid = lax.axis_index("s")
    wid = sid * NC + cid

    pltpu.sync_copy(src_hbm.at[wid], src_v)
    pltpu.sync_copy(dst_hbm.at[wid], dst_v)

    rbase = sid * RPS
    pltpu.sync_copy(g_hbm.at[pl.ds(rbase, RPS)], g_sh.at[pl.ds(rbase, RPS)])
    pltpu.sync_copy(zero_hbm.at[pl.ds(rbase, RPS)], acc_sh.at[pl.ds(rbase, RPS)])
    plsc.subcore_barrier()

    gh = [None, None]
    gh[0] = pltpu.async_copy(g_sh.at[src_v.at[0]], rows_v.at[0], gsem.at[0])
    for j in range(NCHK):
        b = j % 2
        if j + 1 < NCHK:
            gh[1 - b] = pltpu.async_copy(
                g_sh.at[src_v.at[j + 1]], rows_v.at[1 - b], gsem.at[1 - b])
        gh[b].wait()
        pltpu.sync_copy(rows_v.at[b], acc_sh.at[dst_v.at[j]], add=True)

    plsc.subcore_barrier()
    pltpu.sync_copy(acc_sh.at[pl.ds(rbase, RPS)],
                    out_hbm.at[cid, pl.ds(rbase, RPS)])


def _make_sc_spmm(width):
    mesh = plsc.VectorSubcoreMesh(core_axis_name="c", subcore_axis_name="s")
    return pl.kernel(
        _sc_body,
        mesh=mesh,
        compiler_params=pltpu.CompilerParams(use_tc_tiling_on_sc=False),
        out_type=jax.ShapeDtypeStruct((NC, NPAD, width), jnp.float32),
        scratch_types=[
            pltpu.VMEM((NCHK, CH), jnp.int32),
            pltpu.VMEM((NCHK, CH), jnp.int32),
            pltpu.VMEM((2, CH, width), jnp.float32),
            pltpu.VMEM_SHARED((NPAD, width), jnp.float32),
            pltpu.VMEM_SHARED((NPAD, width), jnp.float32),
            pltpu.SemaphoreType.DMA((2,)),
        ],
    )


def _mlp_body(x_ref, w1_ref, b1_ref, w2_ref, b2_ref, o_ref):
    h = jnp.dot(x_ref[...], w1_ref[...], preferred_element_type=jnp.float32)
    h = jnp.maximum(h + b1_ref[...], 0.0)
    o_ref[...] = (
        jnp.dot(h, w2_ref[...], preferred_element_type=jnp.float32) + b2_ref[...]
    )


def _mlp(xp, W1, b1, W2, b2):
    grid = (NPAD // MLP_BLK,)
    return pl.pallas_call(
        _mlp_body,
        grid=grid,
        in_specs=[
            pl.BlockSpec((MLP_BLK, D), lambda i: (i, 0)),
            pl.BlockSpec((D, H), lambda i: (0, 0)),
            pl.BlockSpec((1, H), lambda i: (0, 0)),
            pl.BlockSpec((H, C), lambda i: (0, 0)),
            pl.BlockSpec((1, C), lambda i: (0, 0)),
        ],
        out_specs=pl.BlockSpec((MLP_BLK, C), lambda i: (i, 0)),
        out_shape=jax.ShapeDtypeStruct((NPAD, C), jnp.float32),
    )(xp, W1, b1.reshape(1, H), W2, b2.reshape(1, C))


def _poly_coeff(temp_ref, k):
    pk = 0.0
    for m in range(K + 1):
        b = float(_BMAT[m, k])
        if b != 0.0:
            pk = pk + b * jnp.maximum(temp_ref[m], 0.0)
    return pk


def _norm_body(temp_ref, dacc_ref, h0_ref, dinv_ref, g0_ref, gr_ref):
    deg = dacc_ref[0, :, 0:1] + dacc_ref[1, :, 0:1]
    dinv1 = jnp.where(deg > 0.0, lax.rsqrt(jnp.maximum(deg, 1e-12)), 0.0)
    dinv = jnp.broadcast_to(dinv1, (ROW_BLK, C))
    pK = _poly_coeff(temp_ref, K)
    g0 = dinv * h0_ref[...]
    dinv_ref[...] = dinv
    g0_ref[...] = g0
    gr_ref[...] = pK * g0


def _norm(temp, dacc, h0):
    grid = (NPAD // ROW_BLK,)
    fs = jax.ShapeDtypeStruct((NPAD, C), jnp.float32)
    return pl.pallas_call(
        _norm_body,
        grid=grid,
        in_specs=[
            pl.BlockSpec(memory_space=pltpu.SMEM),
            pl.BlockSpec((2, ROW_BLK, CDEG), lambda i: (0, i, 0)),
            pl.BlockSpec((ROW_BLK, C), lambda i: (i, 0)),
        ],
        out_specs=[pl.BlockSpec((ROW_BLK, C), lambda i: (i, 0))] * 3,
        out_shape=[fs, fs, fs],
    )(temp, dacc, h0)


def _comb_body(temp_ref, acc_ref, g0_ref, dinv_ref, go_ref, *, k):
    dinv = dinv_ref[...]
    pk = _poly_coeff(temp_ref, k)
    go_ref[...] = (dinv * dinv) * (acc_ref[0] + acc_ref[1]) + pk * g0_ref[...]


def _combine(temp, acc, g0, dinv, *, k):
    grid = (NPAD // ROW_BLK,)
    return pl.pallas_call(
        functools.partial(_comb_body, k=k),
        grid=grid,
        in_specs=[
            pl.BlockSpec(memory_space=pltpu.SMEM),
            pl.BlockSpec((2, ROW_BLK, C), lambda i: (0, i, 0)),
            pl.BlockSpec((ROW_BLK, C), lambda i: (i, 0)),
            pl.BlockSpec((ROW_BLK, C), lambda i: (i, 0)),
        ],
        out_specs=pl.BlockSpec((ROW_BLK, C), lambda i: (i, 0)),
        out_shape=jax.ShapeDtypeStruct((NPAD, C), jnp.float32),
    )(temp, acc, g0, dinv)


def _final_body(temp_ref, acc_ref, h0_ref, dinv_ref, o_ref):
    dinv = dinv_ref[...]
    p0 = _poly_coeff(temp_ref, 0)
    r = dinv * (acc_ref[0] + acc_ref[1]) + p0 * h0_ref[...]
    mx = jnp.max(r, axis=1, keepdims=True)
    ex = jnp.exp(r - mx)
    lse = jnp.log(jnp.sum(ex, axis=1, keepdims=True))
    o_ref[...] = r - mx - lse


def _final(temp, acc, h0, dinv):
    grid = (NPAD // ROW_BLK,)
    return pl.pallas_call(
        _final_body,
        grid=grid,
        in_specs=[
            pl.BlockSpec(memory_space=pltpu.SMEM),
            pl.BlockSpec((2, ROW_BLK, C), lambda i: (0, i, 0)),
            pl.BlockSpec((ROW_BLK, C), lambda i: (i, 0)),
            pl.BlockSpec((ROW_BLK, C), lambda i: (i, 0)),
        ],
        out_specs=pl.BlockSpec((ROW_BLK, C), lambda i: (i, 0)),
        out_shape=jax.ShapeDtypeStruct((NPAD, C), jnp.float32),
    )(temp, acc, h0, dinv)


def kernel(x, edge_index, W1, b1, W2, b2, temp):
    xp = jnp.zeros((NPAD, D), jnp.float32).at[:N].set(x)

    pad_e = EPAD - E
    fill = jnp.full((pad_e,), N, jnp.int32)
    srcp = jnp.concatenate([edge_index[0], fill]).reshape(NW, NCHK, CH)
    dstp = jnp.concatenate([edge_index[1], fill]).reshape(NW, NCHK, CH)

    row_valid = (jnp.arange(NPAD, dtype=jnp.int32) < N).astype(jnp.float32)
    ones_deg = jnp.broadcast_to(row_valid[:, None], (NPAD, CDEG))
    zero_feat = jnp.zeros((NPAD, C), jnp.float32)
    zero_deg = jnp.zeros((NPAD, CDEG), jnp.float32)

    sc_spmm = _make_sc_spmm(C)
    sc_deg = _make_sc_spmm(CDEG)

    def spmm(g):
        return sc_spmm(g, srcp, dstp, zero_feat)

    h0 = _mlp(xp, W1, b1, W2, b2)

    dacc = sc_deg(ones_deg, srcp, srcp, zero_deg)
    dinv, g0, g = _norm(temp, dacc, h0)

    for k in range(K - 1, 0, -1):
        acc = spmm(g)
        g = _combine(temp, acc, g0, dinv, k=k)

    acc = spmm(g)
    out = _final(temp, acc, h0, dinv)
    return out[:N]


# speedup vs baseline: 4.9469x; 1.0146x over previous
import functools
import math

import numpy as np
import jax
import jax.numpy as jnp
from jax import lax
from jax.experimental import pallas as pl
from jax.experimental.pallas import tpu as pltpu
from jax.experimental.pallas import tpu_sc as plsc

N = 10000
E = 320000
D = 128
H = 128
C = 64
K = 10

NC = 2
NS = 16
NW = NC * NS

CH = 128
NBUF = 4
LOOKAHEAD = 2
NPAD = 10240
RPS = NPAD // NS
EPAD = ((E + NW * CH * NBUF - 1) // (NW * CH * NBUF)) * (NW * CH * NBUF)
EPT = EPAD // NW
NCHK = EPT // CH

ROW_BLK = 1024
MLP_BLK = 256
CDEG = 16


def _bern_monomial_basis():
    rows = []
    for m in range(K + 1):
        a = np.array([1.0])
        for _ in range(m):
            a = np.convolve(a, [1.0, -1.0])
        for _ in range(K - m):
            a = np.convolve(a, [1.0, 1.0])
        rows.append(math.comb(K, m) / 2.0**K * a)
    return np.stack(rows)


_BMAT = _bern_monomial_basis()


def _sc_body(g_hbm, src_hbm, dst_hbm, zero_hbm, out_hbm,
             src_v, dst_v, rows_v, g_sh, acc_sh, gsem, ssem):
    cid = lax.axis_index("c")
    sid = lax.axis_index("s")
    wid = sid * NC + cid

    pltpu.sync_copy(src_hbm.at[wid], src_v)
    pltpu.sync_copy(dst_hbm.at[wid], dst_v)

    rbase = sid * RPS
    pltpu.sync_copy(g_hbm.at[pl.ds(rbase, RPS)], g_sh.at[pl.ds(rbase, RPS)])
    pltpu.sync_copy(zero_hbm.at[pl.ds(rbase, RPS)], acc_sh.at[pl.ds(rbase, RPS)])
    plsc.subcore_barrier()

    gh = [None, None]
    sh = [None, None]
    gh[0] = pltpu.async_copy(g_sh.at[src_v.at[0]], rows_v.at[0], gsem.at[0])
    for j in range(NCHK):
        b = j % 2
        gh[b].wait()
        sh[b] = pltpu.async_copy(
            rows_v.at[b], acc_sh.at[dst_v.at[j]], ssem.at[b], add=True)
        if j + 1 < NCHK:
            if sh[1 - b] is not None:
                sh[1 - b].wait()
            gh[1 - b] = pltpu.async_copy(
                g_sh.at[src_v.at[j + 1]], rows_v.at[1 - b], gsem.at[1 - b])
    sh[(NCHK - 1) % 2].wait()
    if NCHK > 1:
        sh[NCHK % 2].wait()

    plsc.subcore_barrier()
    pltpu.sync_copy(acc_sh.at[pl.ds(rbase, RPS)],
                    out_hbm.at[cid, pl.ds(rbase, RPS)])


def _make_sc_spmm(width):
    mesh = plsc.VectorSubcoreMesh(core_axis_name="c", subcore_axis_name="s")
    return pl.kernel(
        _sc_body,
        mesh=mesh,
        compiler_params=pltpu.CompilerParams(use_tc_tiling_on_sc=False),
        out_type=jax.ShapeDtypeStruct((NC, NPAD, width), jnp.float32),
        scratch_types=[
            pltpu.VMEM((NCHK, CH), jnp.int32),
            pltpu.VMEM((NCHK, CH), jnp.int32),
            pltpu.VMEM((2, CH, width), jnp.float32),
            pltpu.VMEM_SHARED((NPAD, width), jnp.float32),
            pltpu.VMEM_SHARED((NPAD, width), jnp.float32),
            pltpu.SemaphoreType.DMA((2,)),
            pltpu.SemaphoreType.DMA((2,)),
        ],
    )


def _mlp_body(x_ref, w1_ref, b1_ref, w2_ref, b2_ref, o_ref):
    h = jnp.dot(x_ref[...], w1_ref[...], preferred_element_type=jnp.float32)
    h = jnp.maximum(h + b1_ref[...], 0.0)
    o_ref[...] = (
        jnp.dot(h, w2_ref[...], preferred_element_type=jnp.float32) + b2_ref[...]
    )


def _mlp(xp, W1, b1, W2, b2):
    grid = (NPAD // MLP_BLK,)
    return pl.pallas_call(
        _mlp_body,
        grid=grid,
        in_specs=[
            pl.BlockSpec((MLP_BLK, D), lambda i: (i, 0)),
            pl.BlockSpec((D, H), lambda i: (0, 0)),
            pl.BlockSpec((1, H), lambda i: (0, 0)),
            pl.BlockSpec((H, C), lambda i: (0, 0)),
            pl.BlockSpec((1, C), lambda i: (0, 0)),
        ],
        out_specs=pl.BlockSpec((MLP_BLK, C), lambda i: (i, 0)),
        out_shape=jax.ShapeDtypeStruct((NPAD, C), jnp.float32),
    )(xp, W1, b1.reshape(1, H), W2, b2.reshape(1, C))


def _poly_coeff(temp_ref, k):
    pk = 0.0
    for m in range(K + 1):
        b = float(_BMAT[m, k])
        if b != 0.0:
            pk = pk + b * jnp.maximum(temp_ref[m], 0.0)
    return pk


def _norm_body(temp_ref, dacc_ref, h0_ref, dinv_ref, g0_ref, gr_ref):
    deg = dacc_ref[0, :, 0:1] + dacc_ref[1, :, 0:1]
    dinv1 = jnp.where(deg > 0.0, lax.rsqrt(jnp.maximum(deg, 1e-12)), 0.0)
    dinv = jnp.broadcast_to(dinv1, (ROW_BLK, C))
    pK = _poly_coeff(temp_ref, K)
    g0 = dinv * h0_ref[...]
    dinv_ref[...] = dinv
    g0_ref[...] = g0
    gr_ref[...] = pK * g0


def _norm(temp, dacc, h0):
    grid = (NPAD // ROW_BLK,)
    fs = jax.ShapeDtypeStruct((NPAD, C), jnp.float32)
    return pl.pallas_call(
        _norm_body,
        grid=grid,
        in_specs=[
            pl.BlockSpec(memory_space=pltpu.SMEM),
            pl.BlockSpec((2, ROW_BLK, CDEG), lambda i: (0, i, 0)),
            pl.BlockSpec((ROW_BLK, C), lambda i: (i, 0)),
        ],
        out_specs=[pl.BlockSpec((ROW_BLK, C), lambda i: (i, 0))] * 3,
        out_shape=[fs, fs, fs],
    )(temp, dacc, h0)


def _comb_body(temp_ref, acc_ref, g0_ref, dinv_ref, go_ref, *, k):
    dinv = dinv_ref[...]
    pk = _poly_coeff(temp_ref, k)
    go_ref[...] = (dinv * dinv) * (acc_ref[0] + acc_ref[1]) + pk * g0_ref[...]


def _combine(temp, acc, g0, dinv, *, k):
    grid = (NPAD // ROW_BLK,)
    return pl.pallas_call(
        functools.partial(_comb_body, k=k),
        grid=grid,
        in_specs=[
            pl.BlockSpec(memory_space=pltpu.SMEM),
            pl.BlockSpec((2, ROW_BLK, C), lambda i: (0, i, 0)),
            pl.BlockSpec((ROW_BLK, C), lambda i: (i, 0)),
            pl.BlockSpec((ROW_BLK, C), lambda i: (i, 0)),
        ],
        out_specs=pl.BlockSpec((ROW_BLK, C), lambda i: (i, 0)),
        out_shape=jax.ShapeDtypeStruct((NPAD, C), jnp.float32),
    )(temp, acc, g0, dinv)


def _final_body(temp_ref, acc_ref, h0_ref, dinv_ref, o_ref):
    dinv = dinv_ref[...]
    p0 = _poly_coeff(temp_ref, 0)
    r = dinv * (acc_ref[0] + acc_ref[1]) + p0 * h0_ref[...]
    mx = jnp.max(r, axis=1, keepdims=True)
    ex = jnp.exp(r - mx)
    lse = jnp.log(jnp.sum(ex, axis=1, keepdims=True))
    o_ref[...] = r - mx - lse


def _final(temp, acc, h0, dinv):
    grid = (NPAD // ROW_BLK,)
    return pl.pallas_call(
        _final_body,
        grid=grid,
        in_specs=[
            pl.BlockSpec(memory_space=pltpu.SMEM),
            pl.BlockSpec((2, ROW_BLK, C), lambda i: (0, i, 0)),
            pl.BlockSpec((ROW_BLK, C), lambda i: (i, 0)),
            pl.BlockSpec((ROW_BLK, C), lambda i: (i, 0)),
        ],
        out_specs=pl.BlockSpec((ROW_BLK, C), lambda i: (i, 0)),
        out_shape=jax.ShapeDtypeStruct((NPAD, C), jnp.float32),
    )(temp, acc, h0, dinv)


def kernel(x, edge_index, W1, b1, W2, b2, temp):
    xp = jnp.zeros((NPAD, D), jnp.float32).at[:N].set(x)

    pad_e = EPAD - E
    fill = jnp.full((pad_e,), N, jnp.int32)
    srcp = jnp.concatenate([edge_index[0], fill]).reshape(NW, NCHK, CH)
    dstp = jnp.concatenate([edge_index[1], fill]).reshape(NW, NCHK, CH)

    row_valid = (jnp.arange(NPAD, dtype=jnp.int32) < N).astype(jnp.float32)
    ones_deg = jnp.broadcast_to(row_valid[:, None], (NPAD, CDEG))
    zero_feat = jnp.zeros((NPAD, C), jnp.float32)
    zero_deg = jnp.zeros((NPAD, CDEG), jnp.float32)

    sc_spmm = _make_sc_spmm(C)
    sc_deg = _make_sc_spmm(CDEG)

    def spmm(g):
        return sc_spmm(g, srcp, dstp, zero_feat)

    h0 = _mlp(xp, W1, b1, W2, b2)

    dacc = sc_deg(ones_deg, srcp, srcp, zero_deg)
    dinv, g0, g = _norm(temp, dacc, h0)

    for k in range(K - 1, 0, -1):
        acc = spmm(g)
        g = _combine(temp, acc, g0, dinv, k=k)

    acc = spmm(g)
    out = _final(temp, acc, h0, dinv)
    return out[:N]


# speedup vs baseline: 5.0701x; 1.0249x over previous
import functools
import math

import numpy as np
import jax
import jax.numpy as jnp
from jax import lax
from jax.experimental import pallas as pl
from jax.experimental.pallas import tpu as pltpu
from jax.experimental.pallas import tpu_sc as plsc

N = 10000
E = 320000
D = 128
H = 128
C = 64
K = 10

NC = 2
NS = 16
NW = NC * NS

CH = 128
NBUF = 4
LOOKAHEAD = 2
NPAD = 10240
RPS = NPAD // NS
EPAD = ((E + NW * CH * NBUF - 1) // (NW * CH * NBUF)) * (NW * CH * NBUF)
EPT = EPAD // NW
NCHK = EPT // CH

ROW_BLK = 1024
MLP_BLK = 256
CDEG = 16


def _bern_monomial_basis():
    rows = []
    for m in range(K + 1):
        a = np.array([1.0])
        for _ in range(m):
            a = np.convolve(a, [1.0, -1.0])
        for _ in range(K - m):
            a = np.convolve(a, [1.0, 1.0])
        rows.append(math.comb(K, m) / 2.0**K * a)
    return np.stack(rows)


_BMAT = _bern_monomial_basis()


def _sc_body(g_hbm, src_hbm, dst_hbm, zero_hbm, out_hbm,
             src_v, dst_v, rows_v, g_sh, acc_sh, gsem, ssem):
    cid = lax.axis_index("c")
    sid = lax.axis_index("s")
    wid = sid * NC + cid

    rbase = sid * RPS
    st = [
        pltpu.async_copy(src_hbm.at[wid], src_v, gsem.at[0]),
        pltpu.async_copy(dst_hbm.at[wid], dst_v, gsem.at[1]),
        pltpu.async_copy(g_hbm.at[pl.ds(rbase, RPS)],
                         g_sh.at[pl.ds(rbase, RPS)], ssem.at[0]),
        pltpu.async_copy(zero_hbm.at[pl.ds(rbase, RPS)],
                         acc_sh.at[pl.ds(rbase, RPS)], ssem.at[1]),
    ]
    for h in st:
        h.wait()
    plsc.subcore_barrier()

    gh = [None, None]
    sh = [None, None]
    gh[0] = pltpu.async_copy(g_sh.at[src_v.at[0]], rows_v.at[0], gsem.at[0])
    for j in range(NCHK):
        b = j % 2
        gh[b].wait()
        sh[b] = pltpu.async_copy(
            rows_v.at[b], acc_sh.at[dst_v.at[j]], ssem.at[b], add=True)
        if j + 1 < NCHK:
            if sh[1 - b] is not None:
                sh[1 - b].wait()
            gh[1 - b] = pltpu.async_copy(
                g_sh.at[src_v.at[j + 1]], rows_v.at[1 - b], gsem.at[1 - b])
    sh[(NCHK - 1) % 2].wait()
    if NCHK > 1:
        sh[NCHK % 2].wait()

    plsc.subcore_barrier()
    pltpu.sync_copy(acc_sh.at[pl.ds(rbase, RPS)],
                    out_hbm.at[cid, pl.ds(rbase, RPS)])


def _make_sc_spmm(width):
    mesh = plsc.VectorSubcoreMesh(core_axis_name="c", subcore_axis_name="s")
    return pl.kernel(
        _sc_body,
        mesh=mesh,
        compiler_params=pltpu.CompilerParams(use_tc_tiling_on_sc=False),
        out_type=jax.ShapeDtypeStruct((NC, NPAD, width), jnp.float32),
        scratch_types=[
            pltpu.VMEM((NCHK, CH), jnp.int32),
            pltpu.VMEM((NCHK, CH), jnp.int32),
            pltpu.VMEM((2, CH, width), jnp.float32),
            pltpu.VMEM_SHARED((NPAD, width), jnp.float32),
            pltpu.VMEM_SHARED((NPAD, width), jnp.float32),
            pltpu.SemaphoreType.DMA((2,)),
            pltpu.SemaphoreType.DMA((2,)),
        ],
    )


def _mlp_body(x_ref, w1_ref, b1_ref, w2_ref, b2_ref, o_ref):
    h = jnp.dot(x_ref[...], w1_ref[...], preferred_element_type=jnp.float32)
    h = jnp.maximum(h + b1_ref[...], 0.0)
    o_ref[...] = (
        jnp.dot(h, w2_ref[...], preferred_element_type=jnp.float32) + b2_ref[...]
    )


def _mlp(xp, W1, b1, W2, b2):
    grid = (NPAD // MLP_BLK,)
    return pl.pallas_call(
        _mlp_body,
        grid=grid,
        in_specs=[
            pl.BlockSpec((MLP_BLK, D), lambda i: (i, 0)),
            pl.BlockSpec((D, H), lambda i: (0, 0)),
            pl.BlockSpec((1, H), lambda i: (0, 0)),
            pl.BlockSpec((H, C), lambda i: (0, 0)),
            pl.BlockSpec((1, C), lambda i: (0, 0)),
        ],
        out_specs=pl.BlockSpec((MLP_BLK, C), lambda i: (i, 0)),
        out_shape=jax.ShapeDtypeStruct((NPAD, C), jnp.float32),
    )(xp, W1, b1.reshape(1, H), W2, b2.reshape(1, C))


def _poly_coeff(temp_ref, k):
    pk = 0.0
    for m in range(K + 1):
        b = float(_BMAT[m, k])
        if b != 0.0:
            pk = pk + b * jnp.maximum(temp_ref[m], 0.0)
    return pk


def _norm_body(temp_ref, dacc_ref, h0_ref, dinv_ref, g0_ref, gr_ref):
    deg = dacc_ref[0, :, 0:1] + dacc_ref[1, :, 0:1]
    dinv1 = jnp.where(deg > 0.0, lax.rsqrt(jnp.maximum(deg, 1e-12)), 0.0)
    dinv = jnp.broadcast_to(dinv1, (ROW_BLK, C))
    pK = _poly_coeff(temp_ref, K)
    g0 = dinv * h0_ref[...]
    dinv_ref[...] = dinv
    g0_ref[...] = g0
    gr_ref[...] = pK * g0


def _norm(temp, dacc, h0):
    grid = (NPAD // ROW_BLK,)
    fs = jax.ShapeDtypeStruct((NPAD, C), jnp.float32)
    return pl.pallas_call(
        _norm_body,
        grid=grid,
        in_specs=[
            pl.BlockSpec(memory_space=pltpu.SMEM),
            pl.BlockSpec((2, ROW_BLK, CDEG), lambda i: (0, i, 0)),
            pl.BlockSpec((ROW_BLK, C), lambda i: (i, 0)),
        ],
        out_specs=[pl.BlockSpec((ROW_BLK, C), lambda i: (i, 0))] * 3,
        out_shape=[fs, fs, fs],
    )(temp, dacc, h0)


def _comb_body(temp_ref, acc_ref, g0_ref, dinv_ref, go_ref, *, k):
    dinv = dinv_ref[...]
    pk = _poly_coeff(temp_ref, k)
    go_ref[...] = (dinv * dinv) * (acc_ref[0] + acc_ref[1]) + pk * g0_ref[...]


def _combine(temp, acc, g0, dinv, *, k):
    grid = (NPAD // ROW_BLK,)
    return pl.pallas_call(
        functools.partial(_comb_body, k=k),
        grid=grid,
        in_specs=[
            pl.BlockSpec(memory_space=pltpu.SMEM),
            pl.BlockSpec((2, ROW_BLK, C), lambda i: (0, i, 0)),
            pl.BlockSpec((ROW_BLK, C), lambda i: (i, 0)),
            pl.BlockSpec((ROW_BLK, C), lambda i: (i, 0)),
        ],
        out_specs=pl.BlockSpec((ROW_BLK, C), lambda i: (i, 0)),
        out_shape=jax.ShapeDtypeStruct((NPAD, C), jnp.float32),
    )(temp, acc, g0, dinv)


def _final_body(temp_ref, acc_ref, h0_ref, dinv_ref, o_ref):
    dinv = dinv_ref[...]
    p0 = _poly_coeff(temp_ref, 0)
    r = dinv * (acc_ref[0] + acc_ref[1]) + p0 * h0_ref[...]
    mx = jnp.max(r, axis=1, keepdims=True)
    ex = jnp.exp(r - mx)
    lse = jnp.log(jnp.sum(ex, axis=1, keepdims=True))
    o_ref[...] = r - mx - lse


def _final(temp, acc, h0, dinv):
    grid = (NPAD // ROW_BLK,)
    return pl.pallas_call(
        _final_body,
        grid=grid,
        in_specs=[
            pl.BlockSpec(memory_space=pltpu.SMEM),
            pl.BlockSpec((2, ROW_BLK, C), lambda i: (0, i, 0)),
            pl.BlockSpec((ROW_BLK, C), lambda i: (i, 0)),
            pl.BlockSpec((ROW_BLK, C), lambda i: (i, 0)),
        ],
        out_specs=pl.BlockSpec((ROW_BLK, C), lambda i: (i, 0)),
        out_shape=jax.ShapeDtypeStruct((NPAD, C), jnp.float32),
    )(temp, acc, h0, dinv)


def kernel(x, edge_index, W1, b1, W2, b2, temp):
    xp = jnp.zeros((NPAD, D), jnp.float32).at[:N].set(x)

    pad_e = EPAD - E
    fill = jnp.full((pad_e,), N, jnp.int32)
    srcp = jnp.concatenate([edge_index[0], fill]).reshape(NW, NCHK, CH)
    dstp = jnp.concatenate([edge_index[1], fill]).reshape(NW, NCHK, CH)

    row_valid = (jnp.arange(NPAD, dtype=jnp.int32) < N).astype(jnp.float32)
    ones_deg = jnp.broadcast_to(row_valid[:, None], (NPAD, CDEG))
    zero_feat = jnp.zeros((NPAD, C), jnp.float32)
    zero_deg = jnp.zeros((NPAD, CDEG), jnp.float32)

    sc_spmm = _make_sc_spmm(C)
    sc_deg = _make_sc_spmm(CDEG)

    def spmm(g):
        return sc_spmm(g, srcp, dstp, zero_feat)

    h0 = _mlp(xp, W1, b1, W2, b2)

    dacc = sc_deg(ones_deg, srcp, srcp, zero_deg)
    dinv, g0, g = _norm(temp, dacc, h0)

    for k in range(K - 1, 0, -1):
        acc = spmm(g)
        g = _combine(temp, acc, g0, dinv, k=k)

    acc = spmm(g)
    out = _final(temp, acc, h0, dinv)
    return out[:N]


# speedup vs baseline: 5.0971x; 1.0053x over previous
import functools
import math

import numpy as np
import jax
import jax.numpy as jnp
from jax import lax
from jax.experimental import pallas as pl
from jax.experimental.pallas import tpu as pltpu
from jax.experimental.pallas import tpu_sc as plsc

N = 10000
E = 320000
D = 128
H = 128
C = 64
K = 10

NC = 2
NS = 16
NW = NC * NS

CH = 128
NBUF = 3
NPAD = 10240
RPS = NPAD // NS
EPAD = ((E + NW * CH * 4 - 1) // (NW * CH * 4)) * (NW * CH * 4)
EPT = EPAD // NW
NCHK = EPT // CH

ROW_BLK = 1024
MLP_BLK = 256
CDEG = 16


def _bern_monomial_basis():
    rows = []
    for m in range(K + 1):
        a = np.array([1.0])
        for _ in range(m):
            a = np.convolve(a, [1.0, -1.0])
        for _ in range(K - m):
            a = np.convolve(a, [1.0, 1.0])
        rows.append(math.comb(K, m) / 2.0**K * a)
    return np.stack(rows)


_BMAT = _bern_monomial_basis()


def _sc_body(g_hbm, src_hbm, dst_hbm, zero_hbm, out_hbm,
             src_v, dst_v, rows_v, g_sh, acc_sh, gsem, ssem):
    cid = lax.axis_index("c")
    sid = lax.axis_index("s")
    wid = sid * NC + cid

    rbase = sid * RPS
    st = [
        pltpu.async_copy(src_hbm.at[wid], src_v, gsem.at[0]),
        pltpu.async_copy(dst_hbm.at[wid], dst_v, gsem.at[1]),
        pltpu.async_copy(g_hbm.at[pl.ds(rbase, RPS)],
                         g_sh.at[pl.ds(rbase, RPS)], ssem.at[0]),
        pltpu.async_copy(zero_hbm.at[pl.ds(rbase, RPS)],
                         acc_sh.at[pl.ds(rbase, RPS)], ssem.at[1]),
    ]
    for h in st:
        h.wait()
    plsc.subcore_barrier()

    gh = [None] * NCHK
    sh = [None] * NCHK
    gh[0] = pltpu.async_copy(g_sh.at[src_v.at[0]], rows_v.at[0], gsem.at[0])
    for j in range(NCHK):
        b = j % NBUF
        gh[j].wait()
        sh[j] = pltpu.async_copy(
            rows_v.at[b], acc_sh.at[dst_v.at[j]], ssem.at[b], add=True)
        if j + 1 < NCHK:
            if j + 1 >= NBUF:
                sh[j + 1 - NBUF].wait()
            gh[j + 1] = pltpu.async_copy(
                g_sh.at[src_v.at[j + 1]], rows_v.at[(j + 1) % NBUF],
                gsem.at[(j + 1) % NBUF])
    for j in range(max(0, NCHK - NBUF), NCHK):
        sh[j].wait()

    plsc.subcore_barrier()
    pltpu.sync_copy(acc_sh.at[pl.ds(rbase, RPS)],
                    out_hbm.at[cid, pl.ds(rbase, RPS)])


def _make_sc_spmm(width):
    mesh = plsc.VectorSubcoreMesh(core_axis_name="c", subcore_axis_name="s")
    return pl.kernel(
        _sc_body,
        mesh=mesh,
        compiler_params=pltpu.CompilerParams(use_tc_tiling_on_sc=False),
        out_type=jax.ShapeDtypeStruct((NC, NPAD, width), jnp.float32),
        scratch_types=[
            pltpu.VMEM((NCHK, CH), jnp.int32),
            pltpu.VMEM((NCHK, CH), jnp.int32),
            pltpu.VMEM((NBUF, CH, width), jnp.float32),
            pltpu.VMEM_SHARED((NPAD, width), jnp.float32),
            pltpu.VMEM_SHARED((NPAD, width), jnp.float32),
            pltpu.SemaphoreType.DMA((NBUF,)),
            pltpu.SemaphoreType.DMA((NBUF,)),
        ],
    )


def _mlp_body(x_ref, w1_ref, b1_ref, w2_ref, b2_ref, o_ref):
    h = jnp.dot(x_ref[...], w1_ref[...], preferred_element_type=jnp.float32)
    h = jnp.maximum(h + b1_ref[...], 0.0)
    o_ref[...] = (
        jnp.dot(h, w2_ref[...], preferred_element_type=jnp.float32) + b2_ref[...]
    )


def _mlp(xp, W1, b1, W2, b2):
    grid = (NPAD // MLP_BLK,)
    return pl.pallas_call(
        _mlp_body,
        grid=grid,
        in_specs=[
            pl.BlockSpec((MLP_BLK, D), lambda i: (i, 0)),
            pl.BlockSpec((D, H), lambda i: (0, 0)),
            pl.BlockSpec((1, H), lambda i: (0, 0)),
            pl.BlockSpec((H, C), lambda i: (0, 0)),
            pl.BlockSpec((1, C), lambda i: (0, 0)),
        ],
        out_specs=pl.BlockSpec((MLP_BLK, C), lambda i: (i, 0)),
        out_shape=jax.ShapeDtypeStruct((NPAD, C), jnp.float32),
    )(xp, W1, b1.reshape(1, H), W2, b2.reshape(1, C))


def _poly_coeff(temp_ref, k):
    pk = 0.0
    for m in range(K + 1):
        b = float(_BMAT[m, k])
        if b != 0.0:
            pk = pk + b * jnp.maximum(temp_ref[m], 0.0)
    return pk


def _norm_body(temp_ref, dacc_ref, h0_ref, dinv_ref, g0_ref, gr_ref):
    deg = dacc_ref[0, :, 0:1] + dacc_ref[1, :, 0:1]
    dinv1 = jnp.where(deg > 0.0, lax.rsqrt(jnp.maximum(deg, 1e-12)), 0.0)
    dinv = jnp.broadcast_to(dinv1, (ROW_BLK, C))
    pK = _poly_coeff(temp_ref, K)
    g0 = dinv * h0_ref[...]
    dinv_ref[...] = dinv
    g0_ref[...] = g0
    gr_ref[...] = pK * g0


def _norm(temp, dacc, h0):
    grid = (NPAD // ROW_BLK,)
    fs = jax.ShapeDtypeStruct((NPAD, C), jnp.float32)
    return pl.pallas_call(
        _norm_body,
        grid=grid,
        in_specs=[
            pl.BlockSpec(memory_space=pltpu.SMEM),
            pl.BlockSpec((2, ROW_BLK, CDEG), lambda i: (0, i, 0)),
            pl.BlockSpec((ROW_BLK, C), lambda i: (i, 0)),
        ],
        out_specs=[pl.BlockSpec((ROW_BLK, C), lambda i: (i, 0))] * 3,
        out_shape=[fs, fs, fs],
    )(temp, dacc, h0)


def _comb_body(temp_ref, acc_ref, g0_ref, dinv_ref, go_ref, *, k):
    dinv = dinv_ref[...]
    pk = _poly_coeff(temp_ref, k)
    go_ref[...] = (dinv * dinv) * (acc_ref[0] + acc_ref[1]) + pk * g0_ref[...]


def _combine(temp, acc, g0, dinv, *, k):
    grid = (NPAD // ROW_BLK,)
    return pl.pallas_call(
        functools.partial(_comb_body, k=k),
        grid=grid,
        in_specs=[
            pl.BlockSpec(memory_space=pltpu.SMEM),
            pl.BlockSpec((2, ROW_BLK, C), lambda i: (0, i, 0)),
            pl.BlockSpec((ROW_BLK, C), lambda i: (i, 0)),
            pl.BlockSpec((ROW_BLK, C), lambda i: (i, 0)),
        ],
        out_specs=pl.BlockSpec((ROW_BLK, C), lambda i: (i, 0)),
        out_shape=jax.ShapeDtypeStruct((NPAD, C), jnp.float32),
    )(temp, acc, g0, dinv)


def _final_body(temp_ref, acc_ref, h0_ref, dinv_ref, o_ref):
    dinv = dinv_ref[...]
    p0 = _poly_coeff(temp_ref, 0)
    r = dinv * (acc_ref[0] + acc_ref[1]) + p0 * h0_ref[...]
    mx = jnp.max(r, axis=1, keepdims=True)
    ex = jnp.exp(r - mx)
    lse = jnp.log(jnp.sum(ex, axis=1, keepdims=True))
    o_ref[...] = r - mx - lse


def _final(temp, acc, h0, dinv):
    grid = (NPAD // ROW_BLK,)
    return pl.pallas_call(
        _final_body,
        grid=grid,
        in_specs=[
            pl.BlockSpec(memory_space=pltpu.SMEM),
            pl.BlockSpec((2, ROW_BLK, C), lambda i: (0, i, 0)),
            pl.BlockSpec((ROW_BLK, C), lambda i: (i, 0)),
            pl.BlockSpec((ROW_BLK, C), lambda i: (i, 0)),
        ],
        out_specs=pl.BlockSpec((ROW_BLK, C), lambda i: (i, 0)),
        out_shape=jax.ShapeDtypeStruct((NPAD, C), jnp.float32),
    )(temp, acc, h0, dinv)


def kernel(x, edge_index, W1, b1, W2, b2, temp):
    xp = jnp.zeros((NPAD, D), jnp.float32).at[:N].set(x)

    pad_e = EPAD - E
    fill = jnp.full((pad_e,), N, jnp.int32)
    srcp = jnp.concatenate([edge_index[0], fill]).reshape(NW, NCHK, CH)
    dstp = jnp.concatenate([edge_index[1], fill]).reshape(NW, NCHK, CH)

    row_valid = (jnp.arange(NPAD, dtype=jnp.int32) < N).astype(jnp.float32)
    ones_deg = jnp.broadcast_to(row_valid[:, None], (NPAD, CDEG))
    zero_feat = jnp.zeros((NPAD, C), jnp.float32)
    zero_deg = jnp.zeros((NPAD, CDEG), jnp.float32)

    sc_spmm = _make_sc_spmm(C)
    sc_deg = _make_sc_spmm(CDEG)

    def spmm(g):
        return sc_spmm(g, srcp, dstp, zero_feat)

    h0 = _mlp(xp, W1, b1, W2, b2)

    dacc = sc_deg(ones_deg, srcp, srcp, zero_deg)
    dinv, g0, g = _norm(temp, dacc, h0)

    for k in range(K - 1, 0, -1):
        acc = spmm(g)
        g = _combine(temp, acc, g0, dinv, k=k)

    acc = spmm(g)
    out = _final(temp, acc, h0, dinv)
    return out[:N]
